# Initial kernel scaffold; baseline (speedup 1.0000x reference)
import jax
import jax.numpy as jnp
from jax.experimental import pallas as pl


def kernel(x, edge_index, W_enc, b_enc, W_conv, b_conv, W_ggs, b_ggs, W_ggq, b_ggq, W_skip, W_dec, b_dec):
    raise NotImplementedError("write your pallas kernel here")


# speedup vs baseline: 1.3072x; 1.3072x over previous
import functools

import jax
import jax.numpy as jnp
from jax.experimental import pallas as pl
from jax.experimental.pallas import tpu as pltpu


def _mm_bias_kernel(x_ref, w_ref, b_ref, o_ref, *, act):
    y = jnp.dot(x_ref[...], w_ref[...], preferred_element_type=jnp.float32)
    y = y + b_ref[...]
    if act == "relu":
        y = jnp.maximum(y, 0.0)
    o_ref[...] = y


def _mm_bias(x, w, b, act="none", block_m=1000):
    m, k = x.shape
    n = w.shape[1]
    grid = (m // block_m,)
    return pl.pallas_call(
        functools.partial(_mm_bias_kernel, act=act),
        grid=grid,
        in_specs=[
            pl.BlockSpec((block_m, k), lambda i: (i, 0)),
            pl.BlockSpec((k, n), lambda i: (0, 0)),
            pl.BlockSpec((1, n), lambda i: (0, 0)),
        ],
        out_specs=pl.BlockSpec((block_m, n), lambda i: (i, 0)),
        out_shape=jax.ShapeDtypeStruct((m, n), jnp.float32),
    )(x, w, b.reshape(1, n))


def kernel(x, edge_index, W_enc, b_enc, W_conv, b_conv, W_ggs, b_ggs, W_ggq, b_ggq, W_skip, W_dec, b_dec):
    n = x.shape[0]
    row, col = edge_index[0], edge_index[1]
    ones = jnp.ones(row.shape, jnp.float32)

    indeg = jax.ops.segment_sum(ones, col, num_segments=n)
    outdeg = jax.ops.segment_sum(ones, row, num_segments=n)
    deg = indeg + 1.0
    dis = deg ** -0.5
    cnt = jnp.maximum(outdeg, 1.0)

    X = _mm_bias(x, W_enc, b_enc, act="relu")
    skip_val = _mm_bias(X, W_skip, jnp.zeros((W_skip.shape[1],), jnp.float32))

    for _ in range(2):
        XW = _mm_bias(X, W_conv, jnp.zeros((W_conv.shape[1],), jnp.float32))
        U = dis[:, None] * XW
        agg = jax.ops.segment_sum(U[row], col, num_segments=n)
        conv = dis[:, None] * agg + dis[:, None] ** 2 * XW + b_conv
        X_agg = jnp.maximum(conv, 0.0)

        n2 = jnp.sum(X * X, axis=1)
        S = jax.ops.segment_sum(X[col], row, num_segments=n)
        T = jax.ops.segment_sum(n2[col], row, num_segments=n)
        s = outdeg * n2 + T - 2.0 * jnp.sum(X * S, axis=1)
        gamma = jnp.tanh(s / cnt)[:, None]

        X = (X + gamma * (X_agg + skip_val)) / (1.0 + 2.0 * gamma)

    return _mm_bias(X, W_dec, b_dec)


# speedup vs baseline: 6.7805x; 5.1870x over previous
import functools

import jax
import jax.numpy as jnp
from jax import lax
from jax.experimental import pallas as pl
from jax.experimental.pallas import tpu as pltpu
from jax.experimental.pallas import tpu_sc as plsc

_NC, _NS = 2, 16
_NW = _NC * _NS
_K = 128


def _edge_pass(src, gidx, sidx, scal=None):
    n, W = src.shape
    E = gidx.shape[0]
    n_chunks = E // _K
    align = _NS * 8 * 5
    n_pad = ((n + align - 1) // align) * align
    rows_per_sub = n_pad // _NS
    rows_q = rows_per_sub // 5
    mesh = plsc.VectorSubcoreMesh(core_axis_name="c", subcore_axis_name="s")

    out_type = [jax.ShapeDtypeStruct((_NC, n_pad, W), jnp.float32)]
    scratch = [
        pltpu.VMEM((_K,), jnp.int32),
        pltpu.VMEM((_K,), jnp.int32),
        pltpu.VMEM((_K, W), jnp.float32),
        pltpu.VMEM_SHARED((n_pad, W), jnp.float32),
        pltpu.SemaphoreType.DMA,
    ]
    if scal is not None:
        out_type.append(jax.ShapeDtypeStruct((_NW, n), jnp.float32))
        scratch.append(pltpu.VMEM((n,), jnp.float32))
        scratch.append(pltpu.VMEM((n,), jnp.float32))

    @functools.partial(
        pl.kernel, out_type=out_type, mesh=mesh, scratch_types=scratch,
        compiler_params=pltpu.CompilerParams(needs_layout_passes=False),
    )
    def ep(*refs):
        if scal is not None:
            (src_hbm, gidx_hbm, sidx_hbm, scal_hbm, out_hbm, t_out,
             gi_v, si_v, rows_v, acc_sh, sem, tab_v, t_v) = refs
        else:
            (src_hbm, gidx_hbm, sidx_hbm, out_hbm,
             gi_v, si_v, rows_v, acc_sh, sem) = refs
        cid = lax.axis_index("c")
        sid = lax.axis_index("s")
        wid = sid * _NC + cid
        zv = jnp.zeros((16,), jnp.float32)

        def zrow(i, carry):
            for j in range(W // 16):
                rows_v[i, pl.ds(16 * j, 16)] = zv
            return carry

        lax.fori_loop(0, rows_q, zrow, 0)
        row0 = sid * rows_per_sub
        for t in range(5):
            pltpu.sync_copy(rows_v.at[pl.ds(0, rows_q)], acc_sh.at[pl.ds(row0 + t * rows_q, rows_q)])
        plsc.subcore_barrier()

        if scal is not None:
            pltpu.sync_copy(scal_hbm, tab_v)

            def zt(i, carry):
                t_v[pl.ds(i * 16, 16)] = zv
                return carry

            lax.fori_loop(0, n // 16, zt, 0)

        lanes = lax.iota(jnp.int32, 16)

        def do_chunk(c):
            off = pl.multiple_of(c * _K, _K)
            pltpu.sync_copy(gidx_hbm.at[pl.ds(off, _K)], gi_v)
            pltpu.sync_copy(sidx_hbm.at[pl.ds(off, _K)], si_v)
            pltpu.async_copy(src_hbm.at[gi_v], rows_v, sem).wait()
            pltpu.sync_copy(rows_v, acc_sh.at[si_v], add=True)
            if scal is not None:
                def tgroup(g, carry2):
                    gi16 = gi_v[pl.ds(g * 16, 16)]
                    si16 = si_v[pl.ds(g * 16, 16)]
                    vals = plsc.load_gather(tab_v, [gi16])

                    def lbody(l, carry3):
                        plsc.addupdate_scatter(
                            t_v, [si16], vals, mask=lanes == l)
                        return carry3

                    lax.fori_loop(0, 16, lbody, 0)
                    return carry2

                lax.fori_loop(0, _K // 16, tgroup, 0)

        def body(i, carry):
            do_chunk(wid + _NW * i)
            return carry

        n_full = n_chunks // _NW
        lax.fori_loop(0, n_full, body, 0)
        n_tail = n_chunks - n_full * _NW
        if n_tail:
            @pl.when(wid < n_tail)
            def _():
                do_chunk(n_full * _NW + wid)
        plsc.subcore_barrier()

        for t in range(5):
            r = row0 + t * rows_q
            pltpu.sync_copy(acc_sh.at[pl.ds(r, rows_q)], rows_v.at[pl.ds(0, rows_q)])
            pltpu.sync_copy(rows_v.at[pl.ds(0, rows_q)], out_hbm.at[cid].at[pl.ds(r, rows_q)])
        if scal is not None:
            pltpu.sync_copy(t_v, t_out.at[wid])

    if scal is not None:
        o = ep(src, gidx, sidx, scal)
        return o[0], o[1]
    return ep(src, gidx, sidx)[0]


def _mm_bias_kernel(x_ref, w_ref, b_ref, s_ref, o_ref, *, act, scaled):
    y = jnp.dot(x_ref[...], w_ref[...], preferred_element_type=jnp.float32)
    y = y + b_ref[...]
    if act == "relu":
        y = jnp.maximum(y, 0.0)
    if scaled:
        y = y * s_ref[...]
    o_ref[...] = y


def _mm_bias(x, w, b, act="none", row_scale=None, block_m=1000):
    m, k = x.shape
    n = w.shape[1]
    scaled = row_scale is not None
    if row_scale is None:
        row_scale = jnp.zeros((m, 1), jnp.float32)
    return pl.pallas_call(
        functools.partial(_mm_bias_kernel, act=act, scaled=scaled),
        grid=(m // block_m,),
        in_specs=[
            pl.BlockSpec((block_m, k), lambda i: (i, 0)),
            pl.BlockSpec((k, n), lambda i: (0, 0)),
            pl.BlockSpec((1, n), lambda i: (0, 0)),
            pl.BlockSpec((block_m, 1), lambda i: (i, 0)),
        ],
        out_specs=pl.BlockSpec((block_m, n), lambda i: (i, 0)),
        out_shape=jax.ShapeDtypeStruct((m, n), jnp.float32),
    )(x, w, b.reshape(1, n), row_scale)


def _update_kernel(x_ref, s0_ref, s1_ref, a0_ref, a1_ref, xw_ref, skip_ref,
                   t_ref, od_ref, cnt_ref, dis_ref, bc_ref, o_ref):
    X = x_ref[...]
    S = s0_ref[0] + s1_ref[0]
    agg = a0_ref[0] + a1_ref[0]
    U = xw_ref[...]
    dis = dis_ref[...]
    n2 = jnp.sum(X * X, axis=1, keepdims=True)
    s = od_ref[...] * n2 + t_ref[...] - 2.0 * jnp.sum(X * S, axis=1, keepdims=True)
    gamma = jnp.tanh(s / cnt_ref[...])
    conv = dis * (agg + U) + bc_ref[...]
    X_agg = jnp.maximum(conv, 0.0)
    o_ref[...] = (X + gamma * (X_agg + skip_ref[...])) / (1.0 + 2.0 * gamma)


def _update(X, Sp, Ap, U, skip_val, T, outdeg, cnt, dis, b_conv, block_m=1000):
    m, d = X.shape
    col1 = lambda v: v.reshape(m, 1)
    wide = lambda: pl.BlockSpec((block_m, d), lambda i: (i, 0))
    widep = lambda: pl.BlockSpec((1, block_m, d), lambda i: (0, i, 0))
    narrow = lambda: pl.BlockSpec((block_m, 1), lambda i: (i, 0))
    return pl.pallas_call(
        _update_kernel,
        grid=(m // block_m,),
        in_specs=[wide(), widep(), widep(), widep(), widep(), wide(), wide(),
                  narrow(), narrow(), narrow(), narrow(),
                  pl.BlockSpec((1, d), lambda i: (0, 0))],
        out_specs=wide(),
        out_shape=jax.ShapeDtypeStruct((m, d), jnp.float32),
    )(X, Sp[0:1, :m], Sp[1:2, :m], Ap[0:1, :m], Ap[1:2, :m], U, skip_val,
      col1(T), col1(outdeg), col1(cnt), col1(dis), b_conv.reshape(1, d))


def kernel(x, edge_index, W_enc, b_enc, W_conv, b_conv, W_ggs, b_ggs, W_ggq, b_ggq, W_skip, W_dec, b_dec):
    n = x.shape[0]
    row, col = edge_index[0], edge_index[1]
    ones = jnp.ones(row.shape, jnp.float32)

    indeg = jax.ops.segment_sum(ones, col, num_segments=n)
    outdeg = jax.ops.segment_sum(ones, row, num_segments=n)
    dis = (indeg + 1.0) ** -0.5
    cnt = jnp.maximum(outdeg, 1.0)

    X = _mm_bias(x, W_enc, b_enc, act="relu")
    skip_val = _mm_bias(X, W_skip, jnp.zeros((W_skip.shape[1],), jnp.float32))

    for _ in range(2):
        n2 = jnp.sum(X * X, axis=1)
        Sp, Tp = _edge_pass(X, col, row, scal=n2)
        T = jnp.sum(Tp, axis=0)
        U = _mm_bias(X, W_conv, jnp.zeros((W_conv.shape[1],), jnp.float32),
                     row_scale=dis.reshape(n, 1))
        Ap = _edge_pass(U, row, col)
        X = _update(X, Sp, Ap, U, skip_val, T, outdeg, cnt, dis, b_conv)

    return _mm_bias(X, W_dec, b_dec)


# speedup vs baseline: 8.1978x; 1.2090x over previous
import functools

import jax
import jax.numpy as jnp
from jax import lax
from jax.experimental import pallas as pl
from jax.experimental.pallas import tpu as pltpu
from jax.experimental.pallas import tpu_sc as plsc

_NC, _NS = 2, 16
_NW = _NC * _NS
_K = 128


def _edge_pass(src, gidx, sidx, scal=None, K=128):
    n, W = src.shape
    E = gidx.shape[0]
    n_chunks = E // K
    assert n_chunks * K == E
    align = _NS * 8 * 5
    n_pad = ((n + align - 1) // align) * align
    rows_per_sub = n_pad // _NS
    rows_q = rows_per_sub // 5
    n_full = n_chunks // _NW
    n_tail = n_chunks - n_full * _NW
    mesh = plsc.VectorSubcoreMesh(core_axis_name="c", subcore_axis_name="s")

    out_type = [jax.ShapeDtypeStruct((_NC, n_pad, W), jnp.float32)]
    scratch = [
        pltpu.VMEM((2, K), jnp.int32),
        pltpu.VMEM((2, K), jnp.int32),
        pltpu.VMEM((K, W), jnp.float32),
        pltpu.VMEM((K, W), jnp.float32),
        pltpu.VMEM_SHARED((n_pad, W), jnp.float32),
        pltpu.SemaphoreType.DMA,
        pltpu.SemaphoreType.DMA,
    ]
    if scal is not None:
        out_type.append(jax.ShapeDtypeStruct((_NW, n), jnp.float32))
        scratch.append(pltpu.VMEM((n,), jnp.float32))
        scratch.append(pltpu.VMEM((n,), jnp.float32))

    @functools.partial(
        pl.kernel, out_type=out_type, mesh=mesh, scratch_types=scratch,
        compiler_params=pltpu.CompilerParams(needs_layout_passes=False),
    )
    def ep(*refs):
        if scal is not None:
            (src_hbm, gidx_hbm, sidx_hbm, scal_hbm, out_hbm, t_out,
             gi_v, si_v, rows0, rows1, acc_sh, sem0, sem1, tab_v, t_v) = refs
        else:
            (src_hbm, gidx_hbm, sidx_hbm, out_hbm,
             gi_v, si_v, rows0, rows1, acc_sh, sem0, sem1) = refs
        rows = (rows0, rows1)
        sems = (sem0, sem1)
        cid = lax.axis_index("c")
        sid = lax.axis_index("s")
        wid = sid * _NC + cid
        zv = jnp.zeros((16,), jnp.float32)

        def zrow(i, carry):
            for j in range(W // 16):
                rows0[i, pl.ds(16 * j, 16)] = zv
            return carry

        lax.fori_loop(0, rows_q, zrow, 0)
        row0 = sid * rows_per_sub
        for t in range(5):
            pltpu.sync_copy(rows0.at[pl.ds(0, rows_q)],
                            acc_sh.at[pl.ds(row0 + t * rows_q, rows_q)])
        plsc.subcore_barrier()

        if scal is not None:
            pltpu.sync_copy(scal_hbm, tab_v)

            def zt(i, carry):
                t_v[pl.ds(i * 16, 16)] = zv
                return carry

            lax.fori_loop(0, n // 16, zt, 0)

        lanes = lax.iota(jnp.int32, 16)

        def load_and_start(j, b):
            off = pl.multiple_of((wid + _NW * j) * K, K)
            pltpu.sync_copy(gidx_hbm.at[pl.ds(off, K)], gi_v.at[b])
            pltpu.sync_copy(sidx_hbm.at[pl.ds(off, K)], si_v.at[b])
            pltpu.async_copy(src_hbm.at[gi_v.at[b]], rows[b], sems[b])

        def finish(b):
            pltpu.make_async_copy(src_hbm.at[gi_v.at[b]], rows[b], sems[b]).wait()
            pltpu.sync_copy(rows[b], acc_sh.at[si_v.at[b]], add=True)
            if scal is not None:
                def tgroup(g, carry2):
                    gi16 = gi_v[b, pl.ds(g * 16, 16)]
                    si16 = si_v[b, pl.ds(g * 16, 16)]
                    vals = plsc.load_gather(tab_v, [gi16])

                    def lbody(l, carry3):
                        plsc.addupdate_scatter(
                            t_v, [si16], vals, mask=lanes == l)
                        return carry3

                    lax.fori_loop(0, 16, lbody, 0)
                    return carry2

                lax.fori_loop(0, K // 16, tgroup, 0)

        load_and_start(0, 0)

        def pairbody(k, carry):
            for b in range(2):
                j = 2 * k + b

                @pl.when(j + 1 < n_full)
                def _():
                    load_and_start(j + 1, 1 - b)

                finish(b)
            return carry

        lax.fori_loop(0, n_full // 2, pairbody, 0)
        if n_full % 2:
            finish(0)
        if n_tail:
            @pl.when(wid < n_tail)
            def _():
                off = pl.multiple_of((n_full * _NW + wid) * K, K)
                pltpu.sync_copy(gidx_hbm.at[pl.ds(off, K)], gi_v.at[0])
                pltpu.sync_copy(sidx_hbm.at[pl.ds(off, K)], si_v.at[0])
                pltpu.async_copy(src_hbm.at[gi_v.at[0]], rows[0], sems[0]).wait()
                pltpu.sync_copy(rows[0], acc_sh.at[si_v.at[0]], add=True)
        plsc.subcore_barrier()

        for t in range(5):
            r = row0 + t * rows_q
            pltpu.sync_copy(acc_sh.at[pl.ds(r, rows_q)], rows0.at[pl.ds(0, rows_q)])
            pltpu.sync_copy(rows0.at[pl.ds(0, rows_q)], out_hbm.at[cid].at[pl.ds(r, rows_q)])
        if scal is not None:
            pltpu.sync_copy(t_v, t_out.at[wid])

    if scal is not None:
        o = ep(src, gidx, sidx, scal)
        return o[0], o[1]
    return ep(src, gidx, sidx)[0]


def _mm_bias_kernel(x_ref, w_ref, b_ref, s_ref, o_ref, *, act, scaled):
    y = jnp.dot(x_ref[...], w_ref[...], preferred_element_type=jnp.float32)
    y = y + b_ref[...]
    if act == "relu":
        y = jnp.maximum(y, 0.0)
    if scaled:
        y = y * s_ref[...]
    o_ref[...] = y


def _mm_bias(x, w, b, act="none", row_scale=None, block_m=1000):
    m, k = x.shape
    n = w.shape[1]
    scaled = row_scale is not None
    if row_scale is None:
        row_scale = jnp.zeros((m, 1), jnp.float32)
    return pl.pallas_call(
        functools.partial(_mm_bias_kernel, act=act, scaled=scaled),
        grid=(m // block_m,),
        in_specs=[
            pl.BlockSpec((block_m, k), lambda i: (i, 0)),
            pl.BlockSpec((k, n), lambda i: (0, 0)),
            pl.BlockSpec((1, n), lambda i: (0, 0)),
            pl.BlockSpec((block_m, 1), lambda i: (i, 0)),
        ],
        out_specs=pl.BlockSpec((block_m, n), lambda i: (i, 0)),
        out_shape=jax.ShapeDtypeStruct((m, n), jnp.float32),
    )(x, w, b.reshape(1, n), row_scale)


def _update_kernel(x_ref, s0_ref, s1_ref, a0_ref, a1_ref, xw_ref, skip_ref,
                   t_ref, od_ref, cnt_ref, dis_ref, bc_ref, o_ref):
    X = x_ref[...]
    S = s0_ref[0] + s1_ref[0]
    agg = a0_ref[0] + a1_ref[0]
    U = xw_ref[...]
    dis = dis_ref[...]
    n2 = jnp.sum(X * X, axis=1, keepdims=True)
    s = od_ref[...] * n2 + t_ref[...] - 2.0 * jnp.sum(X * S, axis=1, keepdims=True)
    gamma = jnp.tanh(s / cnt_ref[...])
    conv = dis * (agg + U) + bc_ref[...]
    X_agg = jnp.maximum(conv, 0.0)
    o_ref[...] = (X + gamma * (X_agg + skip_ref[...])) / (1.0 + 2.0 * gamma)


def _update(X, Sp, Ap, U, skip_val, T, outdeg, cnt, dis, b_conv, block_m=1000):
    m, d = X.shape
    col1 = lambda v: v.reshape(m, 1)
    wide = lambda: pl.BlockSpec((block_m, d), lambda i: (i, 0))
    widep = lambda: pl.BlockSpec((1, block_m, d), lambda i: (0, i, 0))
    narrow = lambda: pl.BlockSpec((block_m, 1), lambda i: (i, 0))
    return pl.pallas_call(
        _update_kernel,
        grid=(m // block_m,),
        in_specs=[wide(), widep(), widep(), widep(), widep(), wide(), wide(),
                  narrow(), narrow(), narrow(), narrow(),
                  pl.BlockSpec((1, d), lambda i: (0, 0))],
        out_specs=wide(),
        out_shape=jax.ShapeDtypeStruct((m, d), jnp.float32),
    )(X, Sp[0:1, :m], Sp[1:2, :m], Ap[0:1, :m], Ap[1:2, :m], U, skip_val,
      col1(T), col1(outdeg), col1(cnt), col1(dis), b_conv.reshape(1, d))


def kernel(x, edge_index, W_enc, b_enc, W_conv, b_conv, W_ggs, b_ggs, W_ggq, b_ggq, W_skip, W_dec, b_dec):
    n = x.shape[0]
    row, col = edge_index[0], edge_index[1]
    ones = jnp.ones(row.shape, jnp.float32)

    indeg = jax.ops.segment_sum(ones, col, num_segments=n)
    outdeg = jax.ops.segment_sum(ones, row, num_segments=n)
    dis = (indeg + 1.0) ** -0.5
    cnt = jnp.maximum(outdeg, 1.0)

    X = _mm_bias(x, W_enc, b_enc, act="relu")
    skip_val = _mm_bias(X, W_skip, jnp.zeros((W_skip.shape[1],), jnp.float32))

    for _ in range(2):
        n2 = jnp.sum(X * X, axis=1)
        Sp, Tp = _edge_pass(X, col, row, scal=n2, K=80)
        T = jnp.sum(Tp, axis=0)
        U = _mm_bias(X, W_conv, jnp.zeros((W_conv.shape[1],), jnp.float32),
                     row_scale=dis.reshape(n, 1))
        Ap = _edge_pass(U, row, col)
        X = _update(X, Sp, Ap, U, skip_val, T, outdeg, cnt, dis, b_conv)

    return _mm_bias(X, W_dec, b_dec)
